# baseline (speedup 1.0000x reference)
import functools

import jax
import jax.numpy as jnp
from jax import lax
from jax.experimental import pallas as pl
from jax.experimental.pallas import tpu as pltpu
from jax.experimental.pallas import tpu_sc as plsc

NC = 2
NS = 16
NW = NC * NS
LANES = 16
BLK = 128
CBLKS = 2
CHUNK = BLK * CBLKS
MCAP = 4096
BCAP = 1024
FCAP = 768


def _make_sc_partials(B, D, C):
    full_blocks = C // BLK
    tail_w = C - full_blocks * BLK
    base_blk = full_blocks // NW
    last_blk = full_blocks - (NW - 1) * base_blk
    nchunks = (last_blk + CBLKS - 1) // CBLKS
    tail_bucket = last_blk
    ngroups = B // LANES

    mesh = plsc.VectorSubcoreMesh(core_axis_name="c", subcore_axis_name="s")

    @functools.partial(
        pl.kernel,
        mesh=mesh,
        compiler_params=pltpu.CompilerParams(needs_layout_passes=False),
        out_type=jax.ShapeDtypeStruct((NW, LANES), jnp.float32),
        scratch_types=[
            pltpu.VMEM((B,), jnp.int32),
            pltpu.VMEM((256,), jnp.int32),
            pltpu.VMEM((256,), jnp.int32),
            pltpu.VMEM((256,), jnp.int32),
            pltpu.VMEM((MCAP,), jnp.int32),
            pltpu.VMEM((MCAP,), jnp.int32),
            pltpu.VMEM((D, CHUNK), jnp.float32),
            pltpu.VMEM((D, CHUNK), jnp.float32),
            pltpu.VMEM((FCAP // 2, 2 * D), jnp.float32),
            pltpu.VMEM((D, tail_w), jnp.float32),
            pltpu.VMEM((LANES,), jnp.float32),
            pltpu.VMEM_SHARED((NS, 256), jnp.int32),
            pltpu.VMEM_SHARED((NS, BCAP), jnp.int32),
            pltpu.SMEM((256,), jnp.int32),
            pltpu.SMEM((BCAP,), jnp.int32),
            pltpu.SemaphoreType.DMA,
            pltpu.SemaphoreType.DMA,
            pltpu.SemaphoreType.DMA,
        ],
    )
    def sc_partials(feats_hbm, tgt_hbm, table_hbm, tail_hbm, out_hbm,
                    tgt_v, cnt_v, off_v, placed, matched_t, matched_b,
                    chunk0, chunk1, fstage, tail_v, tot_v,
                    stage_sh, stage_bh, off_s, matched_bs,
                    sem_a, sem_b, sem_f0):
        cid = lax.axis_index("c")
        sid = lax.axis_index("s")
        wid = sid * NC + cid
        is_last = wid == NW - 1

        lo_blk = wid * base_blk
        lo_cls = lo_blk * BLK
        hi_cls = jnp.where(is_last, C, lo_cls + base_blk * BLK)

        pltpu.sync_copy(tgt_hbm, tgt_v)
        pltpu.sync_copy(tail_hbm, tail_v)

        iota = lax.iota(jnp.int32, LANES)
        zeros16i = jnp.zeros((LANES,), jnp.int32)
        ones16i = jnp.ones((LANES,), jnp.int32)

        def zblk(i, _):
            cnt_v[pl.ds(i * LANES, LANES)] = zeros16i
            placed[pl.ds(i * LANES, LANES)] = zeros16i
            return 0

        lax.fori_loop(0, 256 // LANES, zblk, 0)

        def classify(i):
            tv = tgt_v[pl.ds(i * LANES, LANES)]
            m = (tv >= lo_cls) & (tv < hi_cls)
            blk = jnp.where(m, (tv - lo_cls) >> 7, 250)
            return tv, m, blk

        def passa(i, _):
            _, m0, blk0 = classify(2 * i)
            plsc.addupdate_scatter(cnt_v, [blk0], ones16i, mask=m0)
            _, m1, blk1 = classify(2 * i + 1)
            plsc.addupdate_scatter(cnt_v, [blk1], ones16i, mask=m1)
            return 0

        lax.fori_loop(0, ngroups // 2, passa, 0)

        def prefix(i, carry):
            c16 = cnt_v[pl.ds(i * LANES, LANES)]
            inc = plsc.cumsum(c16)
            off_v[pl.ds(i * LANES, LANES)] = inc - c16 + carry
            return carry + jnp.sum(c16)

        lax.fori_loop(0, 256 // LANES, prefix, jnp.int32(0))

        def passb(i, _):
            tv, m, blk = classify(i)
            bv = iota + i * LANES
            cur = plsc.load_gather(placed, [blk])
            rank, _ = plsc.scan_count(blk, mask=m)
            base = plsc.load_gather(off_v, [blk])
            pos = base + cur + rank - 1
            pos = jnp.clip(pos, 0, MCAP - 1)
            plsc.store_scatter(matched_t, [pos], tv, mask=m)
            plsc.store_scatter(matched_b, [pos], bv, mask=m)
            plsc.addupdate_scatter(placed, [blk], ones16i, mask=m)
            return 0

        lax.fori_loop(0, ngroups, passb, 0)

        plsc.subcore_barrier()
        pltpu.sync_copy(off_v, stage_sh.at[sid])
        pltpu.sync_copy(stage_sh.at[sid], off_s)
        pltpu.sync_copy(matched_b.at[pl.ds(0, BCAP)], stage_bh.at[sid])
        pltpu.sync_copy(stage_bh.at[sid], matched_bs)

        total = jnp.zeros((LANES,), jnp.float32)
        bufs = [chunk0, chunk1]
        sems = [sem_a, sem_b]

        def fire(c):
            pltpu.async_copy(
                table_hbm.at[:, pl.ds(lo_cls + c * CHUNK, CHUNK)],
                bufs[c % 2], sems[c % 2],
            )

        n_all = jnp.minimum(off_s[249], FCAP)

        def frow_fire(k, _):
            @pl.when(k < n_all)
            def _():
                b = jnp.clip(matched_bs[jnp.clip(k, 0, BCAP - 1)], 0, B - 1)
                pltpu.async_copy(
                    feats_hbm.at[b],
                    fstage.at[k >> 1, pl.ds((k & 1) * D, D)],
                    sem_f0,
                )

            return 0

        lax.fori_loop(0, FCAP, frow_fire, 0)

        def frow_drain(k, _):
            @pl.when(k < n_all)
            def _():
                pltpu.make_async_copy(
                    feats_hbm.at[0],
                    fstage.at[k >> 1, pl.ds((k & 1) * D, D)],
                    sem_f0,
                ).wait()

            return 0

        lax.fori_loop(0, FCAP, frow_drain, 0)

        def span_compute(js, je, src_buf, col0, total, width=CHUNK):
            def grp(g, tot):
                jb = js + g * LANES
                mask = iota < (je - jb)
                jv = jnp.clip(iota + jb, 0, FCAP - 1)
                tv = plsc.load_gather(matched_t, [jnp.clip(iota + jb, 0, MCAP - 1)])
                colv = jnp.clip(jnp.where(mask, tv - col0, 0), 0, width - 1)
                frow = jv >> 1
                fcol0 = (jv & 1) * D

                def col(d, acc):
                    dv = jnp.full((LANES,), 2 * d, jnp.int32)
                    dv1 = dv + 1
                    cv = plsc.load_gather(src_buf, [dv, colv])
                    fv = plsc.load_gather(fstage, [frow, fcol0 + dv])
                    df = fv - cv
                    cv1 = plsc.load_gather(src_buf, [dv1, colv])
                    fv1 = plsc.load_gather(fstage, [frow, fcol0 + dv1])
                    df1 = fv1 - cv1
                    return acc + df * df + df1 * df1

                acc = lax.fori_loop(
                    0, D // 2, col, jnp.zeros((LANES,), jnp.float32)
                )
                dist = jnp.clip(acc, 1e-12, 1e12)
                return tot + jnp.where(mask, dist, 0.0)

            ng = (je - js + LANES - 1) // LANES
            return lax.fori_loop(0, ng, grp, total)

        def span(c):
            return off_s[c * CBLKS], off_s[(c + 1) * CBLKS]

        def fire_d(c, buf, sem):
            for f in range(D // 8):
                pltpu.async_copy(
                    table_hbm.at[pl.ds(f * 8, 8), pl.ds(lo_cls + c * CHUNK, CHUNK)],
                    buf.at[pl.ds(f * 8, 8)], sem,
                )

        def wait_d(c, buf, sem):
            for f in range(D // 8):
                pltpu.make_async_copy(
                    table_hbm.at[pl.ds(f * 8, 8), pl.ds(lo_cls + c * CHUNK, CHUNK)],
                    buf.at[pl.ds(f * 8, 8)], sem,
                ).wait()

        def do_chunk(c, buf, total):
            js, je = span(c)
            return span_compute(js, je, buf, lo_cls + c * CHUNK, total)

        fire_d(0, chunk0, sem_a)

        def pair(i, total):
            c = 2 * i
            fire_d(c + 1, chunk1, sem_b)
            wait_d(c, chunk0, sem_a)
            total = do_chunk(c, chunk0, total)
            fire_d(c + 2, chunk0, sem_a)
            wait_d(c + 1, chunk1, sem_b)
            return do_chunk(c + 1, chunk1, total)

        total = lax.fori_loop(0, nchunks // 2 - 1, pair, total)
        fire_d(nchunks - 1, chunk1, sem_b)
        wait_d(nchunks - 2, chunk0, sem_a)
        total = do_chunk(nchunks - 2, chunk0, total)
        wait_d(nchunks - 1, chunk1, sem_b)
        total = do_chunk(nchunks - 1, chunk1, total)

        jt0 = off_s[tail_bucket]
        jt1 = jnp.where(is_last, off_s[tail_bucket + 1], jt0)
        total = span_compute(
            jt0, jt1, tail_v, full_blocks * BLK, total, width=tail_w
        )

        tot_v[...] = total
        pltpu.sync_copy(tot_v, out_hbm.at[wid])

    return sc_partials


def kernel(feats, targets, centers):
    B, D = feats.shape
    C = centers.shape[0]
    full_blocks = C // BLK
    tail_start = full_blocks * BLK

    tgt_r = targets.astype(jnp.int32)
    centers_t = centers.T
    tail = centers_t[:, tail_start:]

    partials = _make_sc_partials(B, D, C)(feats, tgt_r, centers_t, tail)

    def tc_reduce(p_ref, o_ref):
        s = 0.5 * jnp.sum(p_ref[...]) * (1.0 / B)
        o_ref[...] = jnp.broadcast_to(s, (1, 1))

    loss = pl.pallas_call(
        tc_reduce,
        out_shape=jax.ShapeDtypeStruct((1, 1), jnp.float32),
    )(partials)
    return loss[0, 0]


# speedup vs baseline: 1.6457x; 1.6457x over previous
import functools

import jax
import jax.numpy as jnp
from jax import lax
from jax.experimental import pallas as pl
from jax.experimental.pallas import tpu as pltpu
from jax.experimental.pallas import tpu_sc as plsc

NC = 2
NS = 16
NW = NC * NS
LANES = 16
BLK = 128
CBLKS = 2
CHUNK = BLK * CBLKS
MCAP = 4096
BCAP = 1024
FCAP = 768


def _make_sc_partials(B, D, C):
    full_blocks = C // BLK
    tail_w = C - full_blocks * BLK
    base_blk = full_blocks // NW
    last_blk = full_blocks - (NW - 1) * base_blk
    nchunks = (last_blk + CBLKS - 1) // CBLKS
    tail_bucket = last_blk
    ngroups = B // LANES

    mesh = plsc.VectorSubcoreMesh(core_axis_name="c", subcore_axis_name="s")

    @functools.partial(
        pl.kernel,
        mesh=mesh,
        compiler_params=pltpu.CompilerParams(needs_layout_passes=False),
        out_type=jax.ShapeDtypeStruct((NW, LANES), jnp.float32),
        scratch_types=[
            pltpu.VMEM((B,), jnp.int32),
            pltpu.VMEM((256,), jnp.int32),
            pltpu.VMEM((256,), jnp.int32),
            pltpu.VMEM((256,), jnp.int32),
            pltpu.VMEM((MCAP,), jnp.int32),
            pltpu.VMEM((MCAP,), jnp.int32),
            pltpu.VMEM((D, CHUNK), jnp.float32),
            pltpu.VMEM((D, CHUNK), jnp.float32),
            pltpu.VMEM((FCAP // 2, 2 * D), jnp.float32),
            pltpu.VMEM((D, tail_w), jnp.float32),
            pltpu.VMEM((LANES,), jnp.float32),
            pltpu.VMEM_SHARED((NS, 256), jnp.int32),
            pltpu.VMEM_SHARED((NS, BCAP), jnp.int32),
            pltpu.SMEM((256,), jnp.int32),
            pltpu.SMEM((BCAP,), jnp.int32),
            pltpu.SemaphoreType.DMA,
            pltpu.SemaphoreType.DMA,
            pltpu.SemaphoreType.DMA,
        ],
    )
    def sc_partials(feats_hbm, tgt_hbm, table_hbm, tail_hbm, out_hbm,
                    tgt_v, cnt_v, off_v, placed, matched_t, matched_b,
                    chunk0, chunk1, fstage, tail_v, tot_v,
                    stage_sh, stage_bh, off_s, matched_bs,
                    sem_a, sem_b, sem_f0):
        cid = lax.axis_index("c")
        sid = lax.axis_index("s")
        wid = sid * NC + cid
        is_last = wid == NW - 1

        lo_blk = wid * base_blk
        lo_cls = lo_blk * BLK
        hi_cls = jnp.where(is_last, C, lo_cls + base_blk * BLK)

        pltpu.sync_copy(tgt_hbm, tgt_v)
        pltpu.sync_copy(tail_hbm, tail_v)

        iota = lax.iota(jnp.int32, LANES)
        zeros16i = jnp.zeros((LANES,), jnp.int32)
        ones16i = jnp.ones((LANES,), jnp.int32)

        def zblk(i, _):
            cnt_v[pl.ds(i * LANES, LANES)] = zeros16i
            placed[pl.ds(i * LANES, LANES)] = zeros16i
            return 0

        lax.fori_loop(0, 256 // LANES, zblk, 0)

        def classify(i):
            tv = tgt_v[pl.ds(i * LANES, LANES)]
            m = (tv >= lo_cls) & (tv < hi_cls)
            blk = jnp.where(m, (tv - lo_cls) >> 7, 250)
            return tv, m, blk

        def passa(i, _):
            _, m0, blk0 = classify(2 * i)
            plsc.addupdate_scatter(cnt_v, [blk0], ones16i, mask=m0)
            _, m1, blk1 = classify(2 * i + 1)
            plsc.addupdate_scatter(cnt_v, [blk1], ones16i, mask=m1)
            return 0

        lax.fori_loop(0, ngroups // 2, passa, 0)

        def prefix(i, carry):
            c16 = cnt_v[pl.ds(i * LANES, LANES)]
            inc = plsc.cumsum(c16)
            off_v[pl.ds(i * LANES, LANES)] = inc - c16 + carry
            return carry + jnp.sum(c16)

        lax.fori_loop(0, 256 // LANES, prefix, jnp.int32(0))

        def passb(i, _):
            tv, m, blk = classify(i)
            bv = iota + i * LANES
            cur = plsc.load_gather(placed, [blk])
            rank, _ = plsc.scan_count(blk, mask=m)
            base = plsc.load_gather(off_v, [blk])
            pos = base + cur + rank - 1
            pos = jnp.clip(pos, 0, MCAP - 1)
            plsc.store_scatter(matched_t, [pos], tv, mask=m)
            plsc.store_scatter(matched_b, [pos], bv, mask=m)
            plsc.addupdate_scatter(placed, [blk], ones16i, mask=m)
            return 0

        lax.fori_loop(0, ngroups, passb, 0)

        plsc.subcore_barrier()
        pltpu.sync_copy(off_v, stage_sh.at[sid])
        pltpu.sync_copy(stage_sh.at[sid], off_s)
        pltpu.sync_copy(matched_b.at[pl.ds(0, BCAP)], stage_bh.at[sid])
        pltpu.sync_copy(stage_bh.at[sid], matched_bs)

        total = jnp.zeros((LANES,), jnp.float32)
        bufs = [chunk0, chunk1]
        sems = [sem_a, sem_b]

        def fire(c):
            pltpu.async_copy(
                table_hbm.at[:, pl.ds(lo_cls + c * CHUNK, CHUNK)],
                bufs[c % 2], sems[c % 2],
            )

        n_all = jnp.minimum(off_s[249], FCAP)

        def frow_fire(k, _):
            @pl.when(k < n_all)
            def _():
                b = jnp.clip(matched_bs[jnp.clip(k, 0, BCAP - 1)], 0, B - 1)
                pltpu.async_copy(
                    feats_hbm.at[b],
                    fstage.at[k >> 1, pl.ds((k & 1) * D, D)],
                    sem_f0,
                )

            return 0

        lax.fori_loop(0, FCAP, frow_fire, 0)

        def frow_drain(k, _):
            @pl.when(k < n_all)
            def _():
                pltpu.make_async_copy(
                    feats_hbm.at[0],
                    fstage.at[k >> 1, pl.ds((k & 1) * D, D)],
                    sem_f0,
                ).wait()

            return 0

        lax.fori_loop(0, FCAP, frow_drain, 0)

        def span_compute(js, je, src_buf, col0, total, width=CHUNK):
            def grp(g, tot):
                jb = js + g * LANES
                mask = iota < (je - jb)
                jv = jnp.clip(iota + jb, 0, FCAP - 1)
                tv = plsc.load_gather(matched_t, [jnp.clip(iota + jb, 0, MCAP - 1)])
                colv = jnp.clip(jnp.where(mask, tv - col0, 0), 0, width - 1)
                frow = jv >> 1
                fcol0 = (jv & 1) * D

                def col(d, acc):
                    dv = jnp.full((LANES,), 2 * d, jnp.int32)
                    dv1 = dv + 1
                    cv = plsc.load_gather(src_buf, [dv, colv])
                    fv = plsc.load_gather(fstage, [frow, fcol0 + dv])
                    df = fv - cv
                    cv1 = plsc.load_gather(src_buf, [dv1, colv])
                    fv1 = plsc.load_gather(fstage, [frow, fcol0 + dv1])
                    df1 = fv1 - cv1
                    return acc + df * df + df1 * df1

                acc = lax.fori_loop(
                    0, D // 2, col, jnp.zeros((LANES,), jnp.float32)
                )
                dist = jnp.clip(acc, 1e-12, 1e12)
                return tot + jnp.where(mask, dist, 0.0)

            ng = (je - js + LANES - 1) // LANES
            return lax.fori_loop(0, ng, grp, total)

        def span(c):
            return off_s[c * CBLKS], off_s[(c + 1) * CBLKS]

        def fire_d(c, buf, sem):
            pltpu.async_copy(
                table_hbm.at[:, pl.ds(lo_cls + c * CHUNK, CHUNK)], buf, sem
            )

        def wait_d(c, buf, sem):
            pltpu.make_async_copy(
                table_hbm.at[:, pl.ds(lo_cls + c * CHUNK, CHUNK)], buf, sem
            ).wait()

        def do_chunk(c, buf, total):
            js, je = span(c)
            return span_compute(js, je, buf, lo_cls + c * CHUNK, total)

        fire_d(0, chunk0, sem_a)

        def pair(i, total):
            c = 2 * i
            fire_d(c + 1, chunk1, sem_b)
            wait_d(c, chunk0, sem_a)
            total = do_chunk(c, chunk0, total)
            fire_d(c + 2, chunk0, sem_a)
            wait_d(c + 1, chunk1, sem_b)
            return do_chunk(c + 1, chunk1, total)

        total = lax.fori_loop(0, nchunks // 2 - 1, pair, total)
        fire_d(nchunks - 1, chunk1, sem_b)
        wait_d(nchunks - 2, chunk0, sem_a)
        total = do_chunk(nchunks - 2, chunk0, total)
        wait_d(nchunks - 1, chunk1, sem_b)
        total = do_chunk(nchunks - 1, chunk1, total)

        jt0 = off_s[tail_bucket]
        jt1 = jnp.where(is_last, off_s[tail_bucket + 1], jt0)
        total = span_compute(
            jt0, jt1, tail_v, full_blocks * BLK, total, width=tail_w
        )

        tot_v[...] = total
        pltpu.sync_copy(tot_v, out_hbm.at[wid])

    return sc_partials


def kernel(feats, targets, centers):
    B, D = feats.shape
    C = centers.shape[0]
    full_blocks = C // BLK
    tail_start = full_blocks * BLK

    tgt_r = targets.astype(jnp.int32)
    centers_t = centers.T
    tail = centers_t[:, tail_start:]

    partials = _make_sc_partials(B, D, C)(feats, tgt_r, centers_t, tail)

    def tc_reduce(p_ref, o_ref):
        s = 0.5 * jnp.sum(p_ref[...]) * (1.0 / B)
        o_ref[...] = jnp.broadcast_to(s, (1, 1))

    loss = pl.pallas_call(
        tc_reduce,
        out_shape=jax.ShapeDtypeStruct((1, 1), jnp.float32),
    )(partials)
    return loss[0, 0]
